# baseline (speedup 1.0000x reference)
import functools

import jax
import jax.numpy as jnp
from jax import lax
from jax.experimental import pallas as pl
from jax.experimental.pallas import tpu as pltpu
from jax.experimental.pallas import tpu_sc as plsc

_B, _T, _D = 4, 2048, 1024
_NC, _NS = 2, 16
_NW = _NC * _NS
_TPW = _T // _NW
_C = 8
_NH = _TPW // _C
_NPAR = 3
_LANES = 16
_VPR = _D // _LANES
_JB = 16

_mesh = plsc.VectorSubcoreMesh(
    core_axis_name="c", subcore_axis_name="s", num_cores=_NC, num_subcores=_NS
)

_scratch_types = [
    pltpu.VMEM((_B * _TPW,), jnp.int32),
    [pltpu.VMEM((_C, _D), jnp.float32) for _ in range(2)],
    [[pltpu.VMEM((_C, _D), jnp.float32) for _ in range(_B)]
     for _ in range(_NPAR)],
    pltpu.SemaphoreType.DMA,
    pltpu.SemaphoreType.DMA,
    pltpu.SemaphoreType.DMA,
    pltpu.SemaphoreType.DMA,
]


def _worker_id():
    return lax.axis_index("s") * _NC + lax.axis_index("c")


def _emb_body(x_hbm, emb_hbm, pos_hbm, out_hbm, idx_v, pos_v, rows_v,
              isem, psem, gsem, osem):
    wid = _worker_id()
    tbase = wid * _TPW

    idx_copies = [
        pltpu.async_copy(
            x_hbm.at[pl.ds(b * _T + tbase, _TPW)],
            idx_v.at[pl.ds(b * _TPW, _TPW)],
            isem,
        )
        for b in range(_B)
    ]

    def start_pos(h):
        return pltpu.async_copy(
            pos_hbm.at[pl.ds(tbase + h * _C, _C)], pos_v[h % 2], psem
        )

    def start_gathers(h):
        par = h % _NPAR
        return [
            pltpu.async_copy(
                emb_hbm.at[idx_v.at[pl.ds(b * _TPW + h * _C, _C)]],
                rows_v[par][b],
                gsem,
            )
            for b in range(_B)
        ]

    def start_outs(h):
        par = h % _NPAR
        return [
            pltpu.async_copy(
                rows_v[par][b],
                out_hbm.at[pl.ds(b * _T + tbase + h * _C, _C)],
                osem,
            )
            for b in range(_B)
        ]

    pending_pos = [start_pos(0)]
    for c in idx_copies:
        c.wait()
    pending_gather = [start_gathers(0)]
    pending_out = []

    for h in range(_NH):
        par = h % _NPAR

        pending_pos.pop(0).wait()
        if h + 1 < _NH:
            pending_pos.append(start_pos(h + 1))

        if h + 1 < _NH:
            if len(pending_out) >= _NPAR - 1:
                for c in pending_out.pop(0):
                    c.wait()
            pending_gather.append(start_gathers(h + 1))

        for c in pending_gather.pop(0):
            c.wait()

        bufs = rows_v[par]
        pv = pos_v[h % 2]

        @plsc.parallel_loop(0, _C * (_VPR // _JB), unroll=2)
        def add_block(i, bufs=bufs, pv=pv):
            r = i // (_VPR // _JB)
            j0 = (i % (_VPR // _JB)) * _JB
            for jj in range(_JB):
                sl = pl.ds((j0 + jj) * _LANES, _LANES)
                p = pv[r, sl]
                for b in range(_B):
                    bufs[b][r, sl] = bufs[b][r, sl] + p

        pending_out.append(start_outs(h))

    while pending_out:
        for c in pending_out.pop(0):
            c.wait()


_emb_kernel = functools.partial(
    pl.kernel,
    out_type=jax.ShapeDtypeStruct((_B * _T, _D), jnp.float32),
    mesh=_mesh,
    scratch_types=_scratch_types,
)(_emb_body)


def kernel(x, embedding, position_embedding):
    xf = x.reshape(-1).astype(jnp.int32)
    out = _emb_kernel(xf, embedding, position_embedding)
    return out.reshape(_B, _T, _D)


# speedup vs baseline: 1.0078x; 1.0078x over previous
import functools

import jax
import jax.numpy as jnp
from jax import lax
from jax.experimental import pallas as pl
from jax.experimental.pallas import tpu as pltpu
from jax.experimental.pallas import tpu_sc as plsc

_B, _T, _D = 4, 2048, 1024
_NC, _NS = 2, 16
_NW = _NC * _NS
_TPW = _T // _NW
_C = 8
_NH = _TPW // _C
_NPAR = 3
_LANES = 16
_VPR = _D // _LANES
_JB = 16

_mesh = plsc.VectorSubcoreMesh(
    core_axis_name="c", subcore_axis_name="s", num_cores=_NC, num_subcores=_NS
)

_scratch_types = [
    pltpu.VMEM((_B, _TPW), jnp.int32),
    [pltpu.VMEM((_C, _D), jnp.float32) for _ in range(2)],
    [pltpu.VMEM((_B, _C, _D), jnp.float32) for _ in range(_NPAR)],
    pltpu.SemaphoreType.DMA,
    pltpu.SemaphoreType.DMA,
    pltpu.SemaphoreType.DMA,
    pltpu.SemaphoreType.DMA,
]


def _worker_id():
    return lax.axis_index("s") * _NC + lax.axis_index("c")


def _emb_body(x_hbm, emb_hbm, pos_hbm, out_hbm, idx_v, pos_v, rows_v,
              isem, psem, gsem, osem):
    wid = _worker_id()
    tbase = wid * _TPW

    idx_copies = [
        pltpu.async_copy(
            x_hbm.at[b, pl.ds(tbase, _TPW)],
            idx_v.at[b],
            isem,
        )
        for b in range(_B)
    ]

    def start_pos(h):
        return pltpu.async_copy(
            pos_hbm.at[pl.ds(tbase + h * _C, _C)], pos_v[h % 2], psem
        )

    def start_gathers(h):
        par = h % _NPAR
        return [
            pltpu.async_copy(
                emb_hbm.at[idx_v.at[b, pl.ds(h * _C, _C)]],
                rows_v[par].at[b],
                gsem,
            )
            for b in range(_B)
        ]

    def start_outs(h):
        par = h % _NPAR
        return [
            pltpu.async_copy(
                rows_v[par],
                out_hbm.at[:, pl.ds(tbase + h * _C, _C), :],
                osem,
            )
        ]

    pending_pos = [start_pos(0)]
    for c in idx_copies:
        c.wait()
    pending_gather = [start_gathers(0)]
    pending_out = []

    for h in range(_NH):
        par = h % _NPAR

        pending_pos.pop(0).wait()
        if h + 1 < _NH:
            pending_pos.append(start_pos(h + 1))

        if h + 1 < _NH:
            if len(pending_out) >= _NPAR - 1:
                for c in pending_out.pop(0):
                    c.wait()
            pending_gather.append(start_gathers(h + 1))

        for c in pending_gather.pop(0):
            c.wait()

        bufs = rows_v[par]
        pv = pos_v[h % 2]

        def add_block(i, carry, bufs=bufs, pv=pv):
            r = i // (_VPR // _JB)
            j0 = (i % (_VPR // _JB)) * _JB
            for jj in range(_JB):
                sl = pl.ds((j0 + jj) * _LANES, _LANES)
                p = pv[r, sl]
                for b in range(_B):
                    bufs[b, r, sl] = bufs[b, r, sl] + p
            return carry

        lax.fori_loop(0, _C * (_VPR // _JB), add_block, 0)

        pending_out.append(start_outs(h))

    while pending_out:
        for c in pending_out.pop(0):
            c.wait()


_emb_kernel = functools.partial(
    pl.kernel,
    out_type=jax.ShapeDtypeStruct((_B, _T, _D), jnp.float32),
    mesh=_mesh,
    scratch_types=_scratch_types,
)(_emb_body)


def kernel(x, embedding, position_embedding):
    xf = x.astype(jnp.int32)
    return _emb_kernel(xf, embedding, position_embedding)


# speedup vs baseline: 1.0954x; 1.0869x over previous
import functools

import jax
import jax.numpy as jnp
from jax import lax
from jax.experimental import pallas as pl
from jax.experimental.pallas import tpu as pltpu
from jax.experimental.pallas import tpu_sc as plsc

_B, _T, _D = 4, 2048, 1024
_NC, _NS = 2, 16
_NW = _NC * _NS
_TPW = _T // _NW
_C = 8
_NH = _TPW // _C
_NPAR = 3
_LANES = 16
_VPR = _D // _LANES
_JB = 16

_mesh = plsc.VectorSubcoreMesh(
    core_axis_name="c", subcore_axis_name="s", num_cores=_NC, num_subcores=_NS
)

_scratch_types = [
    pltpu.VMEM((_B * _TPW,), jnp.int32),
    [pltpu.VMEM((_C, _D), jnp.float32) for _ in range(2)],
    [[pltpu.VMEM((_C, _D), jnp.float32) for _ in range(_B)]
     for _ in range(_NPAR)],
    pltpu.SemaphoreType.DMA,
    pltpu.SemaphoreType.DMA,
    pltpu.SemaphoreType.DMA,
    pltpu.SemaphoreType.DMA,
]


def _worker_id():
    return lax.axis_index("s") * _NC + lax.axis_index("c")


def _emb_body(x_hbm, emb_hbm, pos_hbm, out_hbm, idx_v, pos_v, rows_v,
              isem, psem, gsem, osem):
    wid = _worker_id()
    tbase = wid * _TPW

    idx_copies = [
        pltpu.async_copy(
            x_hbm.at[pl.ds(b * _T + tbase, _TPW)],
            idx_v.at[pl.ds(b * _TPW, _TPW)],
            isem,
        )
        for b in range(_B)
    ]

    def start_pos(h):
        return pltpu.async_copy(
            pos_hbm.at[pl.ds(tbase + h * _C, _C)], pos_v[h % 2], psem
        )

    def start_gathers(h):
        par = h % _NPAR
        return [
            pltpu.async_copy(
                emb_hbm.at[idx_v.at[pl.ds(b * _TPW + h * _C, _C)]],
                rows_v[par][b],
                gsem,
            )
            for b in range(_B)
        ]

    def start_outs(h):
        par = h % _NPAR
        return [
            pltpu.async_copy(
                rows_v[par][b],
                out_hbm.at[pl.ds(b * _T + tbase + h * _C, _C)],
                osem,
            )
            for b in range(_B)
        ]

    pending_pos = [start_pos(0)]
    for c in idx_copies:
        c.wait()
    pending_gather = [start_gathers(0)]
    pending_out = []

    for h in range(_NH):
        par = h % _NPAR

        if h + 1 < _NH:
            if len(pending_out) >= _NPAR - 1:
                for c in pending_out.pop(0):
                    c.wait()
            pending_gather.append(start_gathers(h + 1))

        pending_pos.pop(0).wait()
        if h + 1 < _NH:
            pending_pos.append(start_pos(h + 1))

        for c in pending_gather.pop(0):
            c.wait()

        bufs = rows_v[par]
        pv = pos_v[h % 2]

        def add_block(i, carry, bufs=bufs, pv=pv):
            r = i // (_VPR // _JB)
            j0 = (i % (_VPR // _JB)) * _JB
            for jj in range(_JB):
                sl = pl.ds((j0 + jj) * _LANES, _LANES)
                p = pv[r, sl]
                for b in range(_B):
                    bufs[b][r, sl] = bufs[b][r, sl] + p
            return carry

        lax.fori_loop(0, _C * (_VPR // _JB), add_block, 0)

        pending_out.append(start_outs(h))

    while pending_out:
        for c in pending_out.pop(0):
            c.wait()


_emb_kernel = functools.partial(
    pl.kernel,
    out_type=jax.ShapeDtypeStruct((_B * _T, _D), jnp.float32),
    mesh=_mesh,
    scratch_types=_scratch_types,
)(_emb_body)


def kernel(x, embedding, position_embedding):
    xf = x.reshape(-1).astype(jnp.int32)
    out = _emb_kernel(xf, embedding, position_embedding)
    return out.reshape(_B, _T, _D)
